# Initial kernel scaffold; baseline (speedup 1.0000x reference)
import jax
import jax.numpy as jnp
from jax.experimental import pallas as pl


def kernel(x, edge_index, W_src, W_dst, att_src, att_dst, b_conv, W1, b1, W2, b2):
    raise NotImplementedError("write your pallas kernel here")


# speedup vs baseline: 20.8618x; 20.8618x over previous
import functools

import jax
import jax.numpy as jnp
from jax import lax
from jax.experimental import pallas as pl
from jax.experimental.pallas import tpu as pltpu
from jax.experimental.pallas import tpu_sc as plsc

N = 10000
E = 320000
D = 128
H = 128
O = 128

NC = 2
NS = 16
NW = NC * NS
MB = 128
NB = 79
E_PAD = NW * NB * MB
N_PAD = 10240
ROWS_PER_TILE = N_PAD // NS


def _proj_body(x_ref, ws_ref, wd_ref, as_ref, ad_ref, h_ref, asr_ref, adr_ref):
    x = x_ref[...]
    h = jnp.dot(x, ws_ref[...], preferred_element_type=jnp.float32)
    h_ref[...] = h
    a_s = jnp.dot(h, as_ref[...].T, preferred_element_type=jnp.float32)
    wd_v = jnp.dot(wd_ref[...], ad_ref[...].T, preferred_element_type=jnp.float32)
    a_d = jnp.dot(x, wd_v, preferred_element_type=jnp.float32)
    asr_ref[...] = a_s.reshape(asr_ref.shape)
    adr_ref[...] = a_d.reshape(adr_ref.shape)


def _projections(x, W_src, W_dst, att_src, att_dst):
    nb = 25
    bs = N // nb
    h, a_s, a_d = pl.pallas_call(
        _proj_body,
        grid=(nb,),
        in_specs=[
            pl.BlockSpec((bs, D), lambda i: (i, 0)),
            pl.BlockSpec((D, H), lambda i: (0, 0)),
            pl.BlockSpec((D, H), lambda i: (0, 0)),
            pl.BlockSpec((1, H), lambda i: (0, 0)),
            pl.BlockSpec((1, H), lambda i: (0, 0)),
        ],
        out_specs=[
            pl.BlockSpec((bs, H), lambda i: (i, 0)),
            pl.BlockSpec((1, 1, bs), lambda i: (i, 0, 0)),
            pl.BlockSpec((1, 1, bs), lambda i: (i, 0, 0)),
        ],
        out_shape=[
            jax.ShapeDtypeStruct((N, H), jnp.float32),
            jax.ShapeDtypeStruct((nb, 1, bs), jnp.float32),
            jax.ShapeDtypeStruct((nb, 1, bs), jnp.float32),
        ],
    )(x, W_src, W_dst, att_src.reshape(1, H), att_dst.reshape(1, H))
    return h, a_s.reshape(N), a_d.reshape(N)


def _edge_body(src_hbm, dst_hbm, asrc_hbm, adst_hbm, h_hbm, z2_hbm, z1_hbm,
               numer_out, denom_out,
               asrc_v, adst_v, src_v, dst_v, ex_v, rows_v,
               numer_sh, denom_sh, sem):
    cid = lax.axis_index("c")
    sid = lax.axis_index("s")
    wid = sid * NC + cid

    pltpu.sync_copy(z2_hbm.at[pl.ds(sid * ROWS_PER_TILE, ROWS_PER_TILE), :],
                    numer_sh.at[pl.ds(sid * ROWS_PER_TILE, ROWS_PER_TILE), :])

    @pl.when(sid == 0)
    def _():
        pltpu.sync_copy(z1_hbm, denom_sh)

    pltpu.sync_copy(asrc_hbm, asrc_v)
    pltpu.sync_copy(adst_hbm, adst_v)

    plsc.subcore_barrier()

    def microbatch(b, _):
        base = wid * (NB * MB) + b * MB
        pltpu.sync_copy(src_hbm.at[pl.ds(base, MB)], src_v)
        pltpu.sync_copy(dst_hbm.at[pl.ds(base, MB)], dst_v)

        for g in range(MB // 16):
            si = src_v[pl.ds(g * 16, 16)]
            di = dst_v[pl.ds(g * 16, 16)]
            av = plsc.load_gather(asrc_v, [si])
            bv = plsc.load_gather(adst_v, [di])
            al = av + bv
            al = jnp.where(al >= 0.0, al, 0.2 * al)
            ex_v[pl.ds(g * 16, 16)] = jnp.exp(al)

        pltpu.async_copy(h_hbm.at[src_v], rows_v, sem).wait()

        def scale_group(g, _):
            sv = ex_v[pl.ds(g * 16, 16)]
            for l in range(16):
                s = sv[l]
                r = g * 16 + l
                for c in range(H // 16):
                    sl = pl.ds(c * 16, 16)
                    rows_v[r, sl] = rows_v[r, sl] * s
            return 0

        lax.fori_loop(0, MB // 16, scale_group, 0)

        pltpu.sync_copy(rows_v, numer_sh.at[dst_v], add=True)
        pltpu.sync_copy(ex_v, denom_sh.at[dst_v], add=True)
        return 0

    lax.fori_loop(0, NB, microbatch, 0)

    plsc.subcore_barrier()

    pltpu.sync_copy(numer_sh.at[pl.ds(sid * ROWS_PER_TILE, ROWS_PER_TILE), :],
                    numer_out.at[cid, pl.ds(sid * ROWS_PER_TILE, ROWS_PER_TILE), :])

    @pl.when(sid == 0)
    def _():
        pltpu.sync_copy(denom_sh, denom_out.at[cid])


def _edge_pass(src, dst, a_src, a_dst, h_src, z2, z1):
    mesh = plsc.VectorSubcoreMesh(core_axis_name="c", subcore_axis_name="s")
    return pl.kernel(
        _edge_body,
        out_type=[
            jax.ShapeDtypeStruct((NC, N_PAD, H), jnp.float32),
            jax.ShapeDtypeStruct((NC, N_PAD), jnp.float32),
        ],
        mesh=mesh,
        compiler_params=pltpu.CompilerParams(needs_layout_passes=False),
        scratch_types=[
            pltpu.VMEM((N_PAD,), jnp.float32),
            pltpu.VMEM((N_PAD,), jnp.float32),
            pltpu.VMEM((MB,), jnp.int32),
            pltpu.VMEM((MB,), jnp.int32),
            pltpu.VMEM((MB,), jnp.float32),
            pltpu.VMEM((MB, H), jnp.float32),
            pltpu.VMEM_SHARED((N_PAD, H), jnp.float32),
            pltpu.VMEM_SHARED((N_PAD,), jnp.float32),
            pltpu.SemaphoreType.DMA,
        ],
    )(src, dst, a_src, a_dst, h_src, z2, z1)


def _mlp_body(num_ref, den_ref, bc_ref, w1_ref, b1_ref, w2_ref, b2_ref, out_ref):
    n = num_ref[0] + num_ref[1]
    d = den_ref[0, 0, 0] + den_ref[1, 0, 0]
    h = n / (d[:, None] + 1e-16) + bc_ref[...]
    h = jnp.tanh(h)
    h = jnp.dot(h, w1_ref[...], preferred_element_type=jnp.float32) + b1_ref[...]
    h = jnp.tanh(h)
    out_ref[...] = (jnp.dot(h, w2_ref[...], preferred_element_type=jnp.float32)
                    + b2_ref[...])


def _mlp(numer, denom, b_conv, W1, b1, W2, b2):
    nb = 25
    bs = N // nb
    return pl.pallas_call(
        _mlp_body,
        grid=(nb,),
        in_specs=[
            pl.BlockSpec((NC, bs, H), lambda i: (0, i, 0)),
            pl.BlockSpec((NC, 1, 1, bs), lambda i: (0, i, 0, 0)),
            pl.BlockSpec((1, H), lambda i: (0, 0)),
            pl.BlockSpec((H, H), lambda i: (0, 0)),
            pl.BlockSpec((1, H), lambda i: (0, 0)),
            pl.BlockSpec((H, O), lambda i: (0, 0)),
            pl.BlockSpec((1, O), lambda i: (0, 0)),
        ],
        out_specs=pl.BlockSpec((bs, O), lambda i: (i, 0)),
        out_shape=jax.ShapeDtypeStruct((N, O), jnp.float32),
    )(numer, denom.reshape(NC, nb, 1, bs), b_conv.reshape(1, H), W1,
      b1.reshape(1, H), W2, b2.reshape(1, O))


@jax.jit
def kernel(x, edge_index, W_src, W_dst, att_src, att_dst, b_conv, W1, b1, W2, b2):
    h_src, a_src, a_dst = _projections(x, W_src, W_dst, att_src, att_dst)

    src = jnp.concatenate(
        [edge_index[0].astype(jnp.int32),
         jnp.zeros((E_PAD - E,), jnp.int32)])
    dst = jnp.concatenate(
        [edge_index[1].astype(jnp.int32),
         jnp.full((E_PAD - E,), N, jnp.int32)])
    a_src_p = jnp.concatenate([a_src, jnp.zeros((N_PAD - N,), jnp.float32)])
    a_dst_p = jnp.concatenate([a_dst, jnp.zeros((N_PAD - N,), jnp.float32)])
    z2 = jnp.zeros((N_PAD, H), jnp.float32)
    z1 = jnp.zeros((N_PAD,), jnp.float32)

    numer, denom = _edge_pass(src, dst, a_src_p, a_dst_p, h_src, z2, z1)

    return _mlp(numer[:, :N, :], denom[:, :N], b_conv, W1, b1, W2, b2)
